# baseline (speedup 1.0000x reference)
import dataclasses
import functools

import jax
import jax.numpy as jnp
from jax import lax
from jax.experimental import pallas as pl
from jax.experimental.pallas import tpu as pltpu
from jax.experimental.pallas import tpu_sc as plsc

N_NODES = 10000
N_EDGES = 320000
D = 128
ED = 16
BN_EPS = 1e-5

NC = 2
NS = 16
L = 16
NW = NC * NS
EPW = N_EDGES // NW
C = 16
CH = C // 2
NCHUNK = EPW // C
RPT = 624
TAIL0 = N_NODES - NS * RPT


def _ea_body(attr_t_ref, w_ref, b_ref, o_ref):
    ea = lax.dot_general(
        attr_t_ref[...], w_ref[...],
        dimension_numbers=(((0,), (0,)), ((), ())),
        preferred_element_type=jnp.float32,
    ) + b_ref[...]
    eb = ea.shape[0]
    u = lax.bitcast_convert_type(ea.astype(jnp.bfloat16), jnp.uint16)
    u = u.reshape(eb // C, 2, CH, D).astype(jnp.int32)
    w = (u[:, 1] << 16) | u[:, 0]
    o_ref[...] = w.reshape(eb // 2, D)


def _edge_linear(attr_t, W_edge, b_edge):
    EB = 6400
    return pl.pallas_call(
        _ea_body,
        grid=(N_EDGES // EB,),
        in_specs=[
            pl.BlockSpec((ED, EB), lambda i: (0, i)),
            pl.BlockSpec((ED, D), lambda i: (0, 0)),
            pl.BlockSpec((1, D), lambda i: (0, 0)),
        ],
        out_specs=pl.BlockSpec((EB // 2, D), lambda i: (i, 0)),
        out_shape=jax.ShapeDtypeStruct((N_EDGES // 2, D), jnp.int32),
    )(attr_t, W_edge, b_edge.reshape(1, D))


def _sc_aggregate(x, src, dst, ea, zrows):
    mesh = plsc.VectorSubcoreMesh(core_axis_name="c", subcore_axis_name="s")
    NGRP = (NCHUNK - 1) // 4

    cp = pltpu.CompilerParams()
    if "needs_layout_passes" in pltpu.CompilerParams.__dataclass_fields__:
        cp = dataclasses.replace(cp, needs_layout_passes=False)

    @functools.partial(
        pl.kernel,
        out_type=jax.ShapeDtypeStruct((NC, N_NODES, D), jnp.float32),
        mesh=mesh,
        compiler_params=cp,
        scratch_types=[
            [pltpu.VMEM((C,), jnp.int32) for _ in range(2)],
            [pltpu.VMEM((C,), jnp.int32) for _ in range(4)],
            [pltpu.VMEM((CH, D), jnp.int32) for _ in range(4)],
            [pltpu.VMEM((C, D), jnp.float32) for _ in range(2)],
            [pltpu.VMEM((C, D), jnp.float32) for _ in range(2)],
            pltpu.VMEM_SHARED((N_NODES, D), jnp.float32),
            [pltpu.SemaphoreType.DMA for _ in range(6)],
        ],
    )
    def k(x_hbm, src_hbm, dst_hbm, ea_hbm, z_hbm, out_hbm,
          sidx, didx, eab, xr, msg, aggr_sh, sems):
        sem_in = sems[0:2]
        sem_g = sems[2:4]
        sem_sc = sems[4:6]
        cc = lax.axis_index("c")
        ss = lax.axis_index("s")
        wid = cc * NS + ss
        pltpu.sync_copy(z_hbm.at[pl.ds(0, RPT)], aggr_sh.at[pl.ds(ss * RPT, RPT)])

        @pl.when(ss == 0)
        def _ztail():
            pltpu.sync_copy(z_hbm.at[pl.ds(0, TAIL0)],
                            aggr_sh.at[pl.ds(NS * RPT, TAIL0)])

        plsc.subcore_barrier()

        base_w = wid * EPW

        def in_trips(i, s2, s4):
            base = pl.multiple_of(base_w + i * C, C)
            base2 = pl.multiple_of((base_w + i * C) // 2, CH)
            return (
                (src_hbm.at[pl.ds(base, C)], sidx[s2], sem_in[s2]),
                (dst_hbm.at[pl.ds(base, C)], didx[s4], sem_in[s2]),
                (ea_hbm.at[pl.ds(base2, CH)], eab[s4], sem_in[s2]),
            )

        def issue_in(i, s2, s4):
            for a, b, s in in_trips(i, s2, s4):
                pltpu.async_copy(a, b, s)

        def wait_in(i, s2, s4):
            for a, b, s in in_trips(i, s2, s4):
                pltpu.make_async_copy(a, b, s).wait()

        def issue_g(s2):
            pltpu.async_copy(x_hbm.at[sidx[s2]], xr[s2], sem_g[s2])

        def wait_g(s2):
            pltpu.make_async_copy(x_hbm.at[sidx[s2]], xr[s2], sem_g[s2]).wait()

        def issue_sc(s4, s2):
            pltpu.async_copy(msg[s2], aggr_sh.at[didx[s4]], sem_sc[s2], add=True)

        def wait_sc(s4, s2):
            pltpu.make_async_copy(msg[s2], aggr_sh.at[didx[s4]], sem_sc[s2]).wait()

        def valu(s2, s4):
            @pl.loop(0, CH)
            def _rowpair(q):
                for g in range(D // L):
                    seg = pl.ds(L * g, L)
                    lo, hi = plsc.unpack(
                        plsc.bitcast(eab[s4][q, seg], jnp.bfloat16),
                        format=plsc.PackFormat.INTERLEAVED)
                    msg[s2][q, seg] = jnp.maximum(lo + xr[s2][q, seg], 0.0)
                    msg[s2][q + CH, seg] = jnp.maximum(hi + xr[s2][q + CH, seg], 0.0)

        issue_in(0, 0, 0)
        issue_in(1, 1, 1)
        wait_in(0, 0, 0)
        issue_g(0)

        @pl.loop(0, NGRP)
        def _grp(jg):
            for b in range(4):
                i = jg * 4 + b
                s2 = b % 2
                if b >= 2:
                    wait_sc(b - 2, (b - 2) % 2)
                else:
                    @pl.when(jg > 0)
                    def _wsc():
                        wait_sc((b - 2) % 4, (b - 2) % 2)
                wait_g(s2)
                if b == 3:
                    @pl.when(jg < NGRP - 1)
                    def _pf():
                        issue_in(i + 2, s2, (b + 2) % 4)
                else:
                    issue_in(i + 2, s2, (b + 2) % 4)
                wait_in(i + 1, (b + 1) % 2, (b + 1) % 4)
                issue_g((b + 1) % 2)
                valu(s2, b)
                issue_sc(b, s2)

        wait_sc(2, 0)
        wait_g(0)
        valu(0, 0)
        issue_sc(0, 0)
        wait_sc(3, 1)
        wait_sc(0, 0)

        plsc.subcore_barrier()
        pltpu.sync_copy(
            aggr_sh.at[pl.ds(ss * RPT, RPT)],
            out_hbm.at[cc].at[pl.ds(ss * RPT, RPT)],
        )

        @pl.when(ss == 0)
        def _otail():
            pltpu.sync_copy(
                aggr_sh.at[pl.ds(NS * RPT, TAIL0)],
                out_hbm.at[cc].at[pl.ds(NS * RPT, TAIL0)],
            )

    return k(x, src, dst, ea, zrows)


def _mlp_body(x_ref, a_ref, eps_ref, w1_ref, b1_ref, g1_ref, be1_ref,
              w2_ref, b2_ref, g2_ref, be2_ref, o_ref):
    h = (1.0 + eps_ref[0, 0]) * x_ref[...] + a_ref[0] + a_ref[1]
    h = jnp.dot(h, w1_ref[...], preferred_element_type=jnp.float32) + b1_ref[...]
    mean = jnp.mean(h, axis=0, keepdims=True)
    var = jnp.mean((h - mean) ** 2, axis=0, keepdims=True)
    h = (h - mean) / jnp.sqrt(var + BN_EPS) * g1_ref[...] + be1_ref[...]
    h = jnp.maximum(h, 0.0)
    h = jnp.dot(h, w2_ref[...], preferred_element_type=jnp.float32) + b2_ref[...]
    mean = jnp.mean(h, axis=0, keepdims=True)
    var = jnp.mean((h - mean) ** 2, axis=0, keepdims=True)
    h = (h - mean) / jnp.sqrt(var + BN_EPS) * g2_ref[...] + be2_ref[...]
    o_ref[...] = jnp.maximum(h, 0.0)


def _node_mlp(x, aggr2, eps, W1, b1, g1, be1, W2, b2, g2, be2):
    H = 2 * D
    return pl.pallas_call(
        _mlp_body,
        out_shape=jax.ShapeDtypeStruct((N_NODES, D), jnp.float32),
    )(
        x, aggr2, jnp.reshape(eps, (1, 1)),
        W1, b1.reshape(1, H), g1.reshape(1, H), be1.reshape(1, H),
        W2, b2.reshape(1, D), g2.reshape(1, D), be2.reshape(1, D),
    )


def kernel(x, edge_index, edge_attr_processed, W_edge, b_edge, eps,
           W1, b1, g1, be1, W2, b2, g2, be2):
    src = edge_index[0]
    dst = edge_index[1]
    ea = _edge_linear(edge_attr_processed.T, W_edge, b_edge)
    zrows = jnp.zeros((RPT, D), dtype=jnp.float32)
    aggr2 = _sc_aggregate(x, src, dst, ea, zrows)
    return _node_mlp(x, aggr2, eps, W1, b1, g1, be1, W2, b2, g2, be2)


# speedup vs baseline: 1.8001x; 1.8001x over previous
import functools

import jax
import jax.numpy as jnp
from jax import lax
from jax.experimental import pallas as pl
from jax.experimental.pallas import tpu as pltpu
from jax.experimental.pallas import tpu_sc as plsc

N_NODES = 10000
N_EDGES = 320000
D = 128
ED = 16
BN_EPS = 1e-5

NC = 2
NS = 16
L = 16
NW = NC * NS
EPW = N_EDGES // NW
C = 40
NCHUNK = EPW // C
RPT = 624
TAIL0 = N_NODES - NS * RPT


def _ea_body(attr_t_ref, w_ref, b_ref, o_ref):
    o_ref[...] = lax.dot_general(
        attr_t_ref[...], w_ref[...],
        dimension_numbers=(((0,), (0,)), ((), ())),
        preferred_element_type=jnp.float32,
    ) + b_ref[...]


def _edge_linear(attr_t, W_edge, b_edge):
    EB = 6400
    return pl.pallas_call(
        _ea_body,
        grid=(N_EDGES // EB,),
        in_specs=[
            pl.BlockSpec((ED, EB), lambda i: (0, i)),
            pl.BlockSpec((ED, D), lambda i: (0, 0)),
            pl.BlockSpec((1, D), lambda i: (0, 0)),
        ],
        out_specs=pl.BlockSpec((EB, D), lambda i: (i, 0)),
        out_shape=jax.ShapeDtypeStruct((N_EDGES, D), jnp.float32),
    )(attr_t, W_edge, b_edge.reshape(1, D))


def _sc_aggregate(x, src, dst, ea, zrows):
    mesh = plsc.VectorSubcoreMesh(core_axis_name="c", subcore_axis_name="s")
    NGRP = (NCHUNK - 2) // 4

    @functools.partial(
        pl.kernel,
        out_type=jax.ShapeDtypeStruct((NC, N_NODES, D), jnp.float32),
        mesh=mesh,
        scratch_types=[
            [pltpu.VMEM((C,), jnp.int32) for _ in range(4)],
            [pltpu.VMEM((C,), jnp.int32) for _ in range(4)],
            [pltpu.VMEM((C, D), jnp.float32) for _ in range(4)],
            [pltpu.VMEM((C, D), jnp.float32) for _ in range(4)],
            pltpu.VMEM_SHARED((N_NODES, D), jnp.float32),
            [pltpu.SemaphoreType.DMA for _ in range(8)],
        ],
    )
    def k(x_hbm, src_hbm, dst_hbm, ea_hbm, z_hbm, out_hbm,
          sidx, didx, eab, xr, aggr_sh, sems):
        sem_in = sems[0:2]
        sem_g = sems[2:4]
        sem_sc = sems[4:6]
        sem_d = sems[6:8]
        cc = lax.axis_index("c")
        ss = lax.axis_index("s")
        wid = cc * NS + ss
        pltpu.sync_copy(z_hbm.at[pl.ds(0, RPT)], aggr_sh.at[pl.ds(ss * RPT, RPT)])

        @pl.when(ss == 0)
        def _ztail():
            pltpu.sync_copy(z_hbm.at[pl.ds(0, TAIL0)],
                            aggr_sh.at[pl.ds(NS * RPT, TAIL0)])

        plsc.subcore_barrier()

        base_w = wid * EPW

        def in_pairs(i, s2, s4):
            base = pl.multiple_of(base_w + i * C, C)
            return (
                (src_hbm.at[pl.ds(base, C)], sidx[s4], sem_in[s2]),
                (ea_hbm.at[pl.ds(base, C)], eab[s4], sem_in[s2]),
            )

        def issue_in(i, s2, s4):
            for a, b, s in in_pairs(i, s2, s4):
                pltpu.async_copy(a, b, s)

        def wait_in(i, s2, s4):
            for a, b, s in in_pairs(i, s2, s4):
                pltpu.make_async_copy(a, b, s).wait()

        def d_pair(i, s2, s4):
            base = pl.multiple_of(base_w + i * C, C)
            return (dst_hbm.at[pl.ds(base, C)], didx[s4], sem_d[s2])

        def issue_d(i, s2, s4):
            a, b, s = d_pair(i, s2, s4)
            pltpu.async_copy(a, b, s)

        def wait_d(i, s2, s4):
            a, b, s = d_pair(i, s2, s4)
            pltpu.make_async_copy(a, b, s).wait()

        def issue_g(s2, s4):
            pltpu.async_copy(x_hbm.at[sidx[s4]], xr[s4], sem_g[s2])

        def wait_g(s2, s4):
            pltpu.make_async_copy(x_hbm.at[sidx[s4]], xr[s4], sem_g[s2]).wait()

        def issue_sc(s2, s4):
            pltpu.async_copy(xr[s4], aggr_sh.at[didx[s4]], sem_sc[s2], add=True)

        def wait_sc(s2, s4):
            pltpu.make_async_copy(xr[s4], aggr_sh.at[didx[s4]], sem_sc[s2]).wait()

        def valu(s4):
            @pl.loop(0, C)
            def _row(r):
                for g in range(D // L):
                    seg = pl.ds(L * g, L)
                    xr[s4][r, seg] = jnp.maximum(xr[s4][r, seg] + eab[s4][r, seg], 0.0)

        issue_in(0, 0, 0)
        issue_in(1, 1, 1)
        issue_d(0, 0, 0)
        issue_d(1, 1, 1)
        wait_in(0, 0, 0)
        issue_g(0, 0)
        issue_in(2, 0, 2)
        wait_in(1, 1, 1)
        issue_g(1, 1)

        @pl.loop(0, NGRP)
        def _grp(jg):
            for b in range(4):
                i = jg * 4 + b
                s2 = b % 2
                if b >= 2:
                    wait_sc((b - 2) % 2, b - 2)
                else:
                    @pl.when(jg > 0)
                    def _wsc():
                        wait_sc((b - 2) % 2, (b - 2) % 4)
                wait_g(s2, b)
                wait_d(i, s2, b)
                issue_d(i + 2, s2, (b + 2) % 4)
                if b == 3:
                    @pl.when(jg < NGRP - 1)
                    def _pf():
                        issue_in(i + 3, (b + 3) % 2, (b + 3) % 4)
                else:
                    issue_in(i + 3, (b + 3) % 2, (b + 3) % 4)
                wait_in(i + 2, s2, (b + 2) % 4)
                issue_g(s2, (b + 2) % 4)
                valu(b)
                issue_sc(s2, b)

        wait_sc(0, 2)
        wait_g(0, 0)
        wait_d(NCHUNK - 2, 0, 0)
        valu(0)
        issue_sc(0, 0)
        wait_sc(1, 3)
        wait_g(1, 1)
        wait_d(NCHUNK - 1, 1, 1)
        valu(1)
        issue_sc(1, 1)
        wait_sc(0, 0)
        wait_sc(1, 1)

        plsc.subcore_barrier()
        pltpu.sync_copy(
            aggr_sh.at[pl.ds(ss * RPT, RPT)],
            out_hbm.at[cc].at[pl.ds(ss * RPT, RPT)],
        )

        @pl.when(ss == 0)
        def _otail():
            pltpu.sync_copy(
                aggr_sh.at[pl.ds(NS * RPT, TAIL0)],
                out_hbm.at[cc].at[pl.ds(NS * RPT, TAIL0)],
            )

    return k(x, src, dst, ea, zrows)


def _mlp_body(x_ref, a_ref, eps_ref, w1_ref, b1_ref, g1_ref, be1_ref,
              w2_ref, b2_ref, g2_ref, be2_ref, o_ref):
    h = (1.0 + eps_ref[0, 0]) * x_ref[...] + a_ref[0] + a_ref[1]
    h = jnp.dot(h, w1_ref[...], preferred_element_type=jnp.float32) + b1_ref[...]
    mean = jnp.mean(h, axis=0, keepdims=True)
    var = jnp.mean((h - mean) ** 2, axis=0, keepdims=True)
    h = (h - mean) / jnp.sqrt(var + BN_EPS) * g1_ref[...] + be1_ref[...]
    h = jnp.maximum(h, 0.0)
    h = jnp.dot(h, w2_ref[...], preferred_element_type=jnp.float32) + b2_ref[...]
    mean = jnp.mean(h, axis=0, keepdims=True)
    var = jnp.mean((h - mean) ** 2, axis=0, keepdims=True)
    h = (h - mean) / jnp.sqrt(var + BN_EPS) * g2_ref[...] + be2_ref[...]
    o_ref[...] = jnp.maximum(h, 0.0)


def _node_mlp(x, aggr2, eps, W1, b1, g1, be1, W2, b2, g2, be2):
    H = 2 * D
    return pl.pallas_call(
        _mlp_body,
        out_shape=jax.ShapeDtypeStruct((N_NODES, D), jnp.float32),
    )(
        x, aggr2, jnp.reshape(eps, (1, 1)),
        W1, b1.reshape(1, H), g1.reshape(1, H), be1.reshape(1, H),
        W2, b2.reshape(1, D), g2.reshape(1, D), be2.reshape(1, D),
    )


def kernel(x, edge_index, edge_attr_processed, W_edge, b_edge, eps,
           W1, b1, g1, be1, W2, b2, g2, be2):
    src = edge_index[0]
    dst = edge_index[1]
    ea = _edge_linear(edge_attr_processed.T, W_edge, b_edge)
    zrows = jnp.zeros((RPT, D), dtype=jnp.float32)
    aggr2 = _sc_aggregate(x, src, dst, ea, zrows)
    return _node_mlp(x, aggr2, eps, W1, b1, g1, be1, W2, b2, g2, be2)
